# Initial kernel scaffold; baseline (speedup 1.0000x reference)
import functools

import jax
import jax.numpy as jnp
from jax import lax
from jax.experimental import pallas as pl
from jax.experimental.pallas import tpu as pltpu
from jax.experimental.pallas import tpu_sc as plsc

A = 4
N = 3136
D = 1536
NPAD = 3328
BM = 784
BN = 256
NI = N // BM
NJ = NPAD // BN
NEG = jnp.float32(-3.0e38)


def _blockify(t):
    _, h, w, c = t.shape
    t = t.reshape(1, h // A, A, w // A, A, c)
    t = jnp.moveaxis(t, 2, 3)
    return t.reshape((h // A) * (w // A), A * A * c)


def _unblockify(f, shape):
    _, h, w, c = shape
    t = f.reshape(1, h // A, w // A, A, A, c)
    t = jnp.moveaxis(t, 3, 2)
    return t.reshape(shape)


def _norm_body(ref_in, ref_out):
    v = ref_in[...]
    v = v - jnp.mean(v, axis=1, keepdims=True)
    nrm = jnp.sqrt(jnp.sum(v * v, axis=1, keepdims=True))
    ref_out[...] = v / (nrm + 1e-5)


def _normalize_y(ybp):
    return pl.pallas_call(
        _norm_body,
        grid=(NJ,),
        in_specs=[pl.BlockSpec((BN, D), lambda j: (j, 0))],
        out_specs=pl.BlockSpec((BN, D), lambda j: (j, 0)),
        out_shape=jax.ShapeDtypeStruct((NPAD, D), jnp.float32),
    )(ybp)


def _sim_body(xb_ref, yn_ref, idx_ref, loss_ref, xn_s, rmax_s, ridx_s, acc_s):
    i = pl.program_id(0)
    j = pl.program_id(1)

    @pl.when(j == 0)
    def _():
        xblk = xb_ref[...]
        xc = xblk - jnp.mean(xblk, axis=1, keepdims=True)
        nrm = jnp.sqrt(jnp.sum(xc * xc, axis=1, keepdims=True))
        xn_s[...] = xc / (nrm + 1e-5)
        rmax_s[...] = jnp.full((BM, 1), NEG, jnp.float32)
        ridx_s[...] = jnp.zeros((BM, 1), jnp.int32)

    s = lax.dot_general(
        xn_s[...], yn_ref[...], (((1,), (1,)), ((), ())),
        preferred_element_type=jnp.float32,
    )
    col = j * BN + lax.broadcasted_iota(jnp.int32, (BM, BN), 1)
    s = jnp.where(col < N, s, NEG)
    bmax = jnp.max(s, axis=1, keepdims=True)
    cand = jnp.where(s == bmax, col, jnp.int32(2**31 - 1))
    bidx = jnp.min(cand, axis=1, keepdims=True)
    upd = bmax > rmax_s[...]
    ridx_s[...] = jnp.where(upd, bidx, ridx_s[...])
    rmax_s[...] = jnp.where(upd, bmax, rmax_s[...])

    @pl.when(j == NJ - 1)
    def _():
        idx_ref[...] = ridx_s[...]

        @pl.when(i == 0)
        def _():
            acc_s[0, 0] = 0.0

        acc_s[0, 0] += jnp.sum(1.0 - rmax_s[...])

        @pl.when(i == NI - 1)
        def _():
            loss_ref[0, 0] = acc_s[0, 0] / N


def _sim_argmax(xb, yn):
    return pl.pallas_call(
        _sim_body,
        grid=(NI, NJ),
        in_specs=[
            pl.BlockSpec((BM, D), lambda i, j: (i, 0)),
            pl.BlockSpec((BN, D), lambda i, j: (j, 0)),
        ],
        out_specs=[
            pl.BlockSpec((BM, 1), lambda i, j: (i, 0)),
            pl.BlockSpec((1, 1), lambda i, j: (0, 0)),
        ],
        out_shape=[
            jax.ShapeDtypeStruct((N, 1), jnp.int32),
            jax.ShapeDtypeStruct((1, 1), jnp.float32),
        ],
        scratch_shapes=[
            pltpu.VMEM((BM, D), jnp.float32),
            pltpu.VMEM((BM, 1), jnp.float32),
            pltpu.VMEM((BM, 1), jnp.int32),
            pltpu.SMEM((1, 1), jnp.float32),
        ],
        compiler_params=pltpu.CompilerParams(
            dimension_semantics=("arbitrary", "arbitrary"),
        ),
    )(xb, yn)


def _gather_rows(yn, idxp):
    info = plsc.get_sparse_core_info()
    nw = info.num_cores * info.num_subcores
    bpw = NPAD // nw
    ch = 8
    nch = bpw // ch
    mesh = plsc.VectorSubcoreMesh(core_axis_name="c", subcore_axis_name="s")

    @functools.partial(
        pl.kernel, mesh=mesh,
        out_type=jax.ShapeDtypeStruct((NPAD, D), jnp.float32),
        scratch_types=[
            pltpu.VMEM((bpw,), jnp.int32),
            pltpu.VMEM((ch, D), jnp.float32),
            pltpu.SemaphoreType.DMA,
        ],
    )
    def k(yn_hbm, idx_hbm, out_hbm, idx_v, rows_v, sem):
        wid = lax.axis_index("s") * info.num_cores + lax.axis_index("c")
        base = wid * bpw
        pltpu.sync_copy(idx_hbm.at[pl.ds(base, bpw)], idx_v)
        for c in range(nch):
            pltpu.async_copy(
                yn_hbm.at[idx_v.at[pl.ds(c * ch, ch)]], rows_v, sem
            ).wait()
            pltpu.sync_copy(rows_v, out_hbm.at[pl.ds(base + c * ch, ch)])

    return k(yn, idxp)


def kernel(x, y):
    shape = x.shape
    xb = _blockify(x)
    yb = _blockify(y)
    ybp = jnp.pad(yb, ((0, NPAD - N), (0, 0)))
    yn = _normalize_y(ybp)
    idx2, loss = _sim_argmax(xb, yn)
    idxp = jnp.pad(idx2.reshape(N), (0, NPAD - N))
    newf = _gather_rows(yn, idxp)
    new_x = _unblockify(newf[:N], shape)
    return (loss[0, 0], new_x)


# speedup vs baseline: 1.0750x; 1.0750x over previous
import functools

import jax
import jax.numpy as jnp
from jax import lax
from jax.experimental import pallas as pl
from jax.experimental.pallas import tpu as pltpu
from jax.experimental.pallas import tpu_sc as plsc

A = 4
N = 3136
D = 1536
NPAD = 3328
BM = 784
BN = 256
NI = N // BM
NJ = NPAD // BN
NEG = -3.0e38


def _blockify(t):
    _, h, w, c = t.shape
    t = t.reshape(1, h // A, A, w // A, A, c)
    t = jnp.moveaxis(t, 2, 3)
    return t.reshape((h // A) * (w // A), A * A * c)


def _unblockify(f, shape):
    _, h, w, c = shape
    t = f.reshape(1, h // A, w // A, A, A, c)
    t = jnp.moveaxis(t, 3, 2)
    return t.reshape(shape)


def _norm_body(ref_in, ref_out):
    v = ref_in[...]
    v = v - jnp.mean(v, axis=1, keepdims=True)
    nrm = jnp.sqrt(jnp.sum(v * v, axis=1, keepdims=True))
    ref_out[...] = v / (nrm + 1e-5)


def _normalize_y(ybp):
    return pl.pallas_call(
        _norm_body,
        grid=(NJ,),
        in_specs=[pl.BlockSpec((BN, D), lambda j: (j, 0))],
        out_specs=pl.BlockSpec((BN, D), lambda j: (j, 0)),
        out_shape=jax.ShapeDtypeStruct((NPAD, D), jnp.float32),
    )(ybp)


def _sim_body(xb_ref, yn_ref, idx_ref, loss_ref, xn_s, rmax_s, ridx_s, acc_s):
    i = pl.program_id(0)
    j = pl.program_id(1)

    @pl.when(j == 0)
    def _():
        xblk = xb_ref[...]
        xc = xblk - jnp.mean(xblk, axis=1, keepdims=True)
        nrm = jnp.sqrt(jnp.sum(xc * xc, axis=1, keepdims=True))
        xn_s[...] = xc / (nrm + 1e-5)
        rmax_s[...] = jnp.full((BM, 1), NEG, jnp.float32)
        ridx_s[...] = jnp.zeros((BM, 1), jnp.int32)

    s = lax.dot_general(
        xn_s[...], yn_ref[...], (((1,), (1,)), ((), ())),
        preferred_element_type=jnp.float32,
    )
    col = j * BN + lax.broadcasted_iota(jnp.int32, (BM, BN), 1)
    s = jnp.where(col < N, s, NEG)
    bmax = jnp.max(s, axis=1, keepdims=True)
    cand = jnp.where(s == bmax, col, 2**31 - 1)
    bidx = jnp.min(cand, axis=1, keepdims=True)
    upd = bmax > rmax_s[...]
    ridx_s[...] = jnp.where(upd, bidx, ridx_s[...])
    rmax_s[...] = jnp.where(upd, bmax, rmax_s[...])

    @pl.when(j == NJ - 1)
    def _():
        idx_ref[...] = ridx_s[...]

        @pl.when(i == 0)
        def _():
            acc_s[0, 0] = 0.0

        acc_s[0, 0] += jnp.sum(1.0 - rmax_s[...])

        @pl.when(i == NI - 1)
        def _():
            loss_ref[...] = jnp.full((1, 1), acc_s[0, 0] / N, jnp.float32)


def _sim_argmax(xb, yn):
    return pl.pallas_call(
        _sim_body,
        grid=(NI, NJ),
        in_specs=[
            pl.BlockSpec((BM, D), lambda i, j: (i, 0)),
            pl.BlockSpec((BN, D), lambda i, j: (j, 0)),
        ],
        out_specs=[
            pl.BlockSpec((BM, 1), lambda i, j: (i, 0)),
            pl.BlockSpec((1, 1), lambda i, j: (0, 0)),
        ],
        out_shape=[
            jax.ShapeDtypeStruct((N, 1), jnp.int32),
            jax.ShapeDtypeStruct((1, 1), jnp.float32),
        ],
        scratch_shapes=[
            pltpu.VMEM((BM, D), jnp.float32),
            pltpu.VMEM((BM, 1), jnp.float32),
            pltpu.VMEM((BM, 1), jnp.int32),
            pltpu.SMEM((1, 1), jnp.float32),
        ],
        compiler_params=pltpu.CompilerParams(
            dimension_semantics=("arbitrary", "arbitrary"),
        ),
    )(xb, yn)


def _gather_rows(yn, idxp):
    info = plsc.get_sparse_core_info()
    nw = info.num_cores * info.num_subcores
    bpw = NPAD // nw
    ch = 8
    nch = bpw // ch
    mesh = plsc.VectorSubcoreMesh(core_axis_name="c", subcore_axis_name="s")

    @functools.partial(
        pl.kernel, mesh=mesh,
        out_type=jax.ShapeDtypeStruct((NPAD, D), jnp.float32),
        scratch_types=[
            pltpu.VMEM((bpw,), jnp.int32),
            pltpu.VMEM((ch, D), jnp.float32),
            pltpu.SemaphoreType.DMA,
        ],
    )
    def k(yn_hbm, idx_hbm, out_hbm, idx_v, rows_v, sem):
        wid = lax.axis_index("s") * info.num_cores + lax.axis_index("c")
        base = wid * bpw
        pltpu.sync_copy(idx_hbm.at[pl.ds(base, bpw)], idx_v)
        for c in range(nch):
            pltpu.async_copy(
                yn_hbm.at[idx_v.at[pl.ds(c * ch, ch)]], rows_v, sem
            ).wait()
            pltpu.sync_copy(rows_v, out_hbm.at[pl.ds(base + c * ch, ch)])

    return k(yn, idxp)


def kernel(x, y):
    shape = x.shape
    xb = _blockify(x)
    yb = _blockify(y)
    ybp = jnp.pad(yb, ((0, NPAD - N), (0, 0)))
    yn = _normalize_y(ybp)
    idx2, loss = _sim_argmax(xb, yn)
    idxp = jnp.pad(idx2.reshape(N), (0, NPAD - N))
    newf = _gather_rows(yn, idxp)
    new_x = _unblockify(newf[:N], shape)
    return (loss[0, 0], new_x)


# speedup vs baseline: 1.3468x; 1.2529x over previous
import functools

import jax
import jax.numpy as jnp
from jax import lax
from jax.experimental import pallas as pl
from jax.experimental.pallas import tpu as pltpu
from jax.experimental.pallas import tpu_sc as plsc

A = 4
NB = 56
N = 3136
DS = 384
D = 1536
NPAD = 3328
BM = 784
BN = 448
MI = BM // NB
MJ = BN // NB
NI = N // BM
NJ = N // BN
NEG = -3.0e38


def _strip_view(t):
    return t.reshape(NB, A, NB, DS)


def _assemble_normalize(refs, rows, out_ref):
    parts = [r[...].reshape(rows, DS) for r in refs]
    tot = parts[0] + parts[1] + parts[2] + parts[3]
    mean = jnp.sum(tot, axis=1, keepdims=True) * (1.0 / D)
    sq = None
    for u, p in enumerate(parts):
        c = p - mean
        ps = jnp.sum(c * c, axis=1, keepdims=True)
        sq = ps if sq is None else sq + ps
    inv = 1.0 / (jnp.sqrt(sq) + 1e-5)
    for u, p in enumerate(parts):
        out_ref[:, pl.ds(u * DS, DS)] = (p - mean) * inv


def _ynorm_body(y0, y1, y2, y3, yn_ref):
    _assemble_normalize((y0, y1, y2, y3), BN, yn_ref)


def _normalize_y(y4):
    specs = [
        pl.BlockSpec((MJ, 1, NB, DS), functools.partial(lambda u, j: (j, u, 0, 0), u))
        for u in range(A)
    ]
    return pl.pallas_call(
        _ynorm_body,
        grid=(NJ,),
        in_specs=specs,
        out_specs=pl.BlockSpec((BN, D), lambda j: (j, 0)),
        out_shape=jax.ShapeDtypeStruct((N, D), jnp.float32),
    )(y4, y4, y4, y4)


def _sim_body(x0, x1, x2, x3, yn_ref, idx_ref, loss_ref,
              xn_s, rmax_s, ridx_s, acc_s):
    i = pl.program_id(0)
    j = pl.program_id(1)

    @pl.when(j == 0)
    def _():
        _assemble_normalize((x0, x1, x2, x3), BM, xn_s)
        rmax_s[...] = jnp.full((BM, 1), NEG, jnp.float32)
        ridx_s[...] = jnp.zeros((BM, 1), jnp.int32)

    s = lax.dot_general(
        xn_s[...], yn_ref[...], (((1,), (1,)), ((), ())),
        preferred_element_type=jnp.float32,
    )
    col = j * BN + lax.broadcasted_iota(jnp.int32, (BM, BN), 1)
    bmax = jnp.max(s, axis=1, keepdims=True)
    cand = jnp.where(s == bmax, col, 2**31 - 1)
    bidx = jnp.min(cand, axis=1, keepdims=True)
    upd = bmax > rmax_s[...]
    ridx_s[...] = jnp.where(upd, bidx, ridx_s[...])
    rmax_s[...] = jnp.where(upd, bmax, rmax_s[...])

    @pl.when(j == NJ - 1)
    def _():
        idx_ref[...] = ridx_s[...]

        @pl.when(i == 0)
        def _():
            acc_s[0, 0] = 0.0

        acc_s[0, 0] += jnp.sum(1.0 - rmax_s[...])

        @pl.when(i == NI - 1)
        def _():
            loss_ref[...] = jnp.full((1, 1), acc_s[0, 0] / N, jnp.float32)


def _sim_argmax(x4, yn):
    xspecs = [
        pl.BlockSpec((MI, 1, NB, DS),
                     functools.partial(lambda u, i, j: (i, u, 0, 0), u))
        for u in range(A)
    ]
    return pl.pallas_call(
        _sim_body,
        grid=(NI, NJ),
        in_specs=xspecs + [pl.BlockSpec((BN, D), lambda i, j: (j, 0))],
        out_specs=[
            pl.BlockSpec((BM, 1), lambda i, j: (i, 0)),
            pl.BlockSpec((1, 1), lambda i, j: (0, 0)),
        ],
        out_shape=[
            jax.ShapeDtypeStruct((N, 1), jnp.int32),
            jax.ShapeDtypeStruct((1, 1), jnp.float32),
        ],
        scratch_shapes=[
            pltpu.VMEM((BM, D), jnp.float32),
            pltpu.VMEM((BM, 1), jnp.float32),
            pltpu.VMEM((BM, 1), jnp.int32),
            pltpu.SMEM((1, 1), jnp.float32),
        ],
        compiler_params=pltpu.CompilerParams(
            dimension_semantics=("arbitrary", "arbitrary"),
        ),
    )(x4, x4, x4, x4, yn)


def _gather_rows(yn, idxp):
    info = plsc.get_sparse_core_info()
    nw = info.num_cores * info.num_subcores
    bpw = NPAD // nw
    ch = 8
    nch = bpw // ch
    mesh = plsc.VectorSubcoreMesh(core_axis_name="c", subcore_axis_name="s")

    @functools.partial(
        pl.kernel, mesh=mesh,
        out_type=jax.ShapeDtypeStruct((NPAD, D), jnp.float32),
        scratch_types=[
            pltpu.VMEM((bpw,), jnp.int32),
            pltpu.VMEM((ch, D), jnp.float32),
            pltpu.SemaphoreType.DMA,
        ],
    )
    def k(yn_hbm, idx_hbm, out_hbm, idx_v, rows_v, sem):
        wid = lax.axis_index("s") * info.num_cores + lax.axis_index("c")
        base = wid * bpw
        pltpu.sync_copy(idx_hbm.at[pl.ds(base, bpw)], idx_v)
        for c in range(nch):
            pltpu.async_copy(
                yn_hbm.at[idx_v.at[pl.ds(c * ch, ch)]], rows_v, sem
            ).wait()
            pltpu.sync_copy(rows_v, out_hbm.at[pl.ds(base + c * ch, ch)])

    return k(yn, idxp)


def kernel(x, y):
    shape = x.shape
    x4 = _strip_view(x)
    y4 = _strip_view(y)
    yn = _normalize_y(y4)
    idx2, loss = _sim_argmax(x4, yn)
    idxp = jnp.pad(idx2.reshape(N), (0, NPAD - N))
    newf = _gather_rows(yn, idxp)[:N]
    t = newf.reshape(NB, NB, A, DS)
    new_x = jnp.moveaxis(t, 2, 1).reshape(shape)
    return (loss[0, 0], new_x)


# speedup vs baseline: 1.5882x; 1.1792x over previous
import functools

import jax
import jax.numpy as jnp
from jax import lax
from jax.experimental import pallas as pl
from jax.experimental.pallas import tpu as pltpu
from jax.experimental.pallas import tpu_sc as plsc

A = 4
NB = 56
N = 3136
DS = 384
D = 1536
NPAD = 3328
BM = 784
BN = 448
MI = BM // NB
MJ = BN // NB
NI = N // BM
NJ = N // BN
NEG = -3.0e38


def _strip_view(t):
    return t.reshape(NB, A, NB, DS)


def _assemble_normalize(refs, rows, out_ref):
    parts = [r[...].reshape(rows, DS) for r in refs]
    tot = parts[0] + parts[1] + parts[2] + parts[3]
    mean = jnp.sum(tot, axis=1, keepdims=True) * (1.0 / D)
    sq = None
    for u, p in enumerate(parts):
        c = p - mean
        ps = jnp.sum(c * c, axis=1, keepdims=True)
        sq = ps if sq is None else sq + ps
    inv = 1.0 / (jnp.sqrt(sq) + 1e-5)
    for u, p in enumerate(parts):
        out_ref[:, pl.ds(u * DS, DS)] = (p - mean) * inv


def _ynorm_body(y0, y1, y2, y3, yn_ref):
    _assemble_normalize((y0, y1, y2, y3), BN, yn_ref)


def _normalize_y(y4):
    specs = [
        pl.BlockSpec((MJ, 1, NB, DS), functools.partial(lambda u, j: (j, u, 0, 0), u))
        for u in range(A)
    ]
    return pl.pallas_call(
        _ynorm_body,
        grid=(NJ,),
        in_specs=specs,
        out_specs=pl.BlockSpec((BN, D), lambda j: (j, 0)),
        out_shape=jax.ShapeDtypeStruct((N, D), jnp.float32),
    )(y4, y4, y4, y4)


def _sim_body(x0, x1, x2, x3, yn_ref, idx_ref, loss_ref,
              xn_s, rmax_s, ridx_s, acc_s):
    i = pl.program_id(0)
    j = pl.program_id(1)

    @pl.when(j == 0)
    def _():
        _assemble_normalize((x0, x1, x2, x3), BM, xn_s)
        rmax_s[...] = jnp.full((BM, 1), NEG, jnp.float32)
        ridx_s[...] = jnp.zeros((BM, 1), jnp.int32)

    s = lax.dot_general(
        xn_s[...], yn_ref[...], (((1,), (1,)), ((), ())),
        preferred_element_type=jnp.float32,
    )
    col = j * BN + lax.broadcasted_iota(jnp.int32, (BM, BN), 1)
    bmax = jnp.max(s, axis=1, keepdims=True)
    cand = jnp.where(s == bmax, col, 2**31 - 1)
    bidx = jnp.min(cand, axis=1, keepdims=True)
    upd = bmax > rmax_s[...]
    ridx_s[...] = jnp.where(upd, bidx, ridx_s[...])
    rmax_s[...] = jnp.where(upd, bmax, rmax_s[...])

    @pl.when(j == NJ - 1)
    def _():
        idx_ref[...] = ridx_s[...]

        @pl.when(i == 0)
        def _():
            acc_s[0, 0] = 0.0

        acc_s[0, 0] += jnp.sum(1.0 - rmax_s[...])

        @pl.when(i == NI - 1)
        def _():
            loss_ref[...] = jnp.full((1, 1), acc_s[0, 0] / N, jnp.float32)


def _sim_argmax(x4, yn):
    xspecs = [
        pl.BlockSpec((MI, 1, NB, DS),
                     functools.partial(lambda u, i, j: (i, u, 0, 0), u))
        for u in range(A)
    ]
    return pl.pallas_call(
        _sim_body,
        grid=(NI, NJ),
        in_specs=xspecs + [pl.BlockSpec((BN, D), lambda i, j: (j, 0))],
        out_specs=[
            pl.BlockSpec((BM, 1), lambda i, j: (i, 0)),
            pl.BlockSpec((1, 1), lambda i, j: (0, 0)),
        ],
        out_shape=[
            jax.ShapeDtypeStruct((N, 1), jnp.int32),
            jax.ShapeDtypeStruct((1, 1), jnp.float32),
        ],
        scratch_shapes=[
            pltpu.VMEM((BM, D), jnp.float32),
            pltpu.VMEM((BM, 1), jnp.float32),
            pltpu.VMEM((BM, 1), jnp.int32),
            pltpu.SMEM((1, 1), jnp.float32),
        ],
        compiler_params=pltpu.CompilerParams(
            dimension_semantics=("arbitrary", "arbitrary"),
        ),
    )(x4, x4, x4, x4, yn)


def _gather_rows(yn, idxp):
    info = plsc.get_sparse_core_info()
    nw = 28
    bpw = N // nw
    ch = 8
    nch = bpw // ch
    mesh = plsc.VectorSubcoreMesh(core_axis_name="c", subcore_axis_name="s")

    @functools.partial(
        pl.kernel, mesh=mesh,
        out_type=jax.ShapeDtypeStruct((NB, A, NB, DS), jnp.float32),
        scratch_types=[
            pltpu.VMEM((bpw,), jnp.int32),
            pltpu.VMEM((ch, D), jnp.float32),
            pltpu.VMEM((ch, D), jnp.float32),
            pltpu.SemaphoreType.DMA,
            pltpu.SemaphoreType.DMA,
            pltpu.SemaphoreType.DMA,
        ],
    )
    def k(yn_hbm, idx_hbm, out_hbm, idx_v, buf0, buf1, gsem0, gsem1, wsem):
        wid = lax.axis_index("s") * info.num_cores + lax.axis_index("c")

        @pl.when(wid < nw)
        def _():
            base = wid * bpw
            pltpu.sync_copy(idx_hbm.at[pl.ds(base, bpw)], idx_v)
            bufs = (buf0, buf1)
            gsems = (gsem0, gsem1)
            gathers = [None] * nch
            writes = [[] for _ in range(nch)]
            gathers[0] = pltpu.async_copy(
                yn_hbm.at[idx_v.at[pl.ds(0, ch)]], bufs[0], gsems[0])
            for c in range(nch):
                cur = bufs[c % 2]
                gathers[c].wait()
                if c + 1 < nch:
                    if c >= 1:
                        for h in writes[c - 1]:
                            h.wait()
                    gathers[c + 1] = pltpu.async_copy(
                        yn_hbm.at[idx_v.at[pl.ds((c + 1) * ch, ch)]],
                        bufs[(c + 1) % 2], gsems[(c + 1) % 2])
                bi = 2 * wid + (c // 7)
                bj = ch * (c % 7)
                for u in range(A):
                    writes[c].append(pltpu.make_async_copy(
                        cur.at[:, pl.ds(u * DS, DS)],
                        out_hbm.at[bi, u, pl.ds(bj, ch)],
                        wsem))
                    writes[c][-1].start()
            for c in (nch - 2, nch - 1):
                for h in writes[c]:
                    h.wait()

    return k(yn, idxp)


def kernel(x, y):
    shape = x.shape
    x4 = _strip_view(x)
    y4 = _strip_view(y)
    yn = _normalize_y(y4)
    idx2, loss = _sim_argmax(x4, yn)
    new_x = _gather_rows(yn, idx2.reshape(N)).reshape(shape)
    return (loss[0, 0], new_x)


# speedup vs baseline: 1.5927x; 1.0028x over previous
import functools

import jax
import jax.numpy as jnp
from jax import lax
from jax.experimental import pallas as pl
from jax.experimental.pallas import tpu as pltpu
from jax.experimental.pallas import tpu_sc as plsc

A = 4
NB = 56
N = 3136
DS = 384
D = 1536
NPAD = 3328
BM = 784
BN = 448
MI = BM // NB
MJ = BN // NB
NI = N // BM
NJ = N // BN
NEG = -3.0e38


def _strip_view(t):
    return t.reshape(NB, A, NB, DS)


def _assemble_normalize(ref, rows, out_ref):
    parts = [ref[:, u].reshape(rows, DS) for u in range(A)]
    tot = parts[0] + parts[1] + parts[2] + parts[3]
    mean = jnp.sum(tot, axis=1, keepdims=True) * (1.0 / D)
    sq = None
    for u, p in enumerate(parts):
        c = p - mean
        ps = jnp.sum(c * c, axis=1, keepdims=True)
        sq = ps if sq is None else sq + ps
    inv = 1.0 / (jnp.sqrt(sq) + 1e-5)
    for u, p in enumerate(parts):
        out_ref[:, pl.ds(u * DS, DS)] = (p - mean) * inv


def _ynorm_body(y_ref, yn_ref):
    _assemble_normalize(y_ref, BN, yn_ref)


def _normalize_y(y4):
    return pl.pallas_call(
        _ynorm_body,
        grid=(NJ,),
        in_specs=[pl.BlockSpec((MJ, A, NB, DS), lambda j: (j, 0, 0, 0))],
        out_specs=pl.BlockSpec((BN, D), lambda j: (j, 0)),
        out_shape=jax.ShapeDtypeStruct((N, D), jnp.float32),
    )(y4)


def _sim_body(x_ref, yn_ref, idx_ref, loss_ref,
              xn_s, rmax_s, ridx_s, acc_s):
    i = pl.program_id(0)
    j = pl.program_id(1)

    @pl.when(j == 0)
    def _():
        _assemble_normalize(x_ref, BM, xn_s)
        rmax_s[...] = jnp.full((BM, 1), NEG, jnp.float32)
        ridx_s[...] = jnp.zeros((BM, 1), jnp.int32)

    s = lax.dot_general(
        xn_s[...], yn_ref[...], (((1,), (1,)), ((), ())),
        preferred_element_type=jnp.float32,
    )
    col = j * BN + lax.broadcasted_iota(jnp.int32, (BM, BN), 1)
    bmax = jnp.max(s, axis=1, keepdims=True)
    cand = jnp.where(s == bmax, col, 2**31 - 1)
    bidx = jnp.min(cand, axis=1, keepdims=True)
    upd = bmax > rmax_s[...]
    ridx_s[...] = jnp.where(upd, bidx, ridx_s[...])
    rmax_s[...] = jnp.where(upd, bmax, rmax_s[...])

    @pl.when(j == NJ - 1)
    def _():
        idx_ref[...] = ridx_s[...]

        @pl.when(i == 0)
        def _():
            acc_s[0, 0] = 0.0

        acc_s[0, 0] += jnp.sum(1.0 - rmax_s[...])

        @pl.when(i == NI - 1)
        def _():
            loss_ref[...] = jnp.full((1, 1), acc_s[0, 0] / N, jnp.float32)


def _sim_argmax(x4, yn):
    return pl.pallas_call(
        _sim_body,
        grid=(NI, NJ),
        in_specs=[
            pl.BlockSpec((MI, A, NB, DS), lambda i, j: (i, 0, 0, 0)),
            pl.BlockSpec((BN, D), lambda i, j: (j, 0)),
        ],
        out_specs=[
            pl.BlockSpec((BM, 1), lambda i, j: (i, 0)),
            pl.BlockSpec((1, 1), lambda i, j: (0, 0)),
        ],
        out_shape=[
            jax.ShapeDtypeStruct((N, 1), jnp.int32),
            jax.ShapeDtypeStruct((1, 1), jnp.float32),
        ],
        scratch_shapes=[
            pltpu.VMEM((BM, D), jnp.float32),
            pltpu.VMEM((BM, 1), jnp.float32),
            pltpu.VMEM((BM, 1), jnp.int32),
            pltpu.SMEM((1, 1), jnp.float32),
        ],
        compiler_params=pltpu.CompilerParams(
            dimension_semantics=("arbitrary", "arbitrary"),
        ),
    )(x4, yn)


def _gather_rows(yn, idxp):
    info = plsc.get_sparse_core_info()
    nw = 28
    bpw = N // nw
    ch = 8
    nch = bpw // ch
    mesh = plsc.VectorSubcoreMesh(core_axis_name="c", subcore_axis_name="s")

    @functools.partial(
        pl.kernel, mesh=mesh,
        out_type=jax.ShapeDtypeStruct((NB, A, NB, DS), jnp.float32),
        scratch_types=[
            pltpu.VMEM((bpw,), jnp.int32),
            pltpu.VMEM((ch, D), jnp.float32),
            pltpu.VMEM((ch, D), jnp.float32),
            pltpu.SemaphoreType.DMA,
            pltpu.SemaphoreType.DMA,
            pltpu.SemaphoreType.DMA,
        ],
    )
    def k(yn_hbm, idx_hbm, out_hbm, idx_v, buf0, buf1, gsem0, gsem1, wsem):
        wid = lax.axis_index("s") * info.num_cores + lax.axis_index("c")

        @pl.when(wid < nw)
        def _():
            base = wid * bpw
            pltpu.sync_copy(idx_hbm.at[pl.ds(base, bpw)], idx_v)
            bufs = (buf0, buf1)
            gsems = (gsem0, gsem1)
            gathers = [None] * nch
            writes = [[] for _ in range(nch)]
            gathers[0] = pltpu.async_copy(
                yn_hbm.at[idx_v.at[pl.ds(0, ch)]], bufs[0], gsems[0])
            for c in range(nch):
                cur = bufs[c % 2]
                gathers[c].wait()
                if c + 1 < nch:
                    if c >= 1:
                        for h in writes[c - 1]:
                            h.wait()
                    gathers[c + 1] = pltpu.async_copy(
                        yn_hbm.at[idx_v.at[pl.ds((c + 1) * ch, ch)]],
                        bufs[(c + 1) % 2], gsems[(c + 1) % 2])
                bi = 2 * wid + (c // 7)
                bj = ch * (c % 7)
                for u in range(A):
                    writes[c].append(pltpu.make_async_copy(
                        cur.at[:, pl.ds(u * DS, DS)],
                        out_hbm.at[bi, u, pl.ds(bj, ch)],
                        wsem))
                    writes[c][-1].start()
            for c in (nch - 2, nch - 1):
                for h in writes[c]:
                    h.wait()

    return k(yn, idxp)


def kernel(x, y):
    shape = x.shape
    x4 = _strip_view(x)
    y4 = _strip_view(y)
    yn = _normalize_y(y4)
    idx2, loss = _sim_argmax(x4, yn)
    new_x = _gather_rows(yn, idx2.reshape(N)).reshape(shape)
    return (loss[0, 0], new_x)


# speedup vs baseline: 2.2304x; 1.4003x over previous
import functools

import jax
import jax.numpy as jnp
from jax import lax
from jax.experimental import pallas as pl
from jax.experimental.pallas import tpu as pltpu
from jax.experimental.pallas import tpu_sc as plsc

A = 4
NB = 56
IMW = 224
C = 96
N = 3136
DS = 384
D = 1536
NPAD = 3328
BM = 784
BN = 448
MI = BM // NB
MJ = BN // NB
NI = N // BM
NJ = N // BN
NEG = -3.0e38


def _assemble_normalize(ref, s_ref, rows, out_ref):
    g = rows // NB
    val = ref[0].reshape(A * g * C, IMW)
    q = lax.dot_general(val, s_ref[...], (((1,), (0,)), ((), ())),
                        precision=lax.Precision.HIGHEST,
                        preferred_element_type=jnp.float32)
    q = q.reshape(g, A, C, IMW)
    qt = jnp.transpose(q, (0, 1, 3, 2))
    qt = qt.reshape(g, A, A, NB, C)
    parts = [qt[:, u, v].reshape(rows, C)
             for u in range(A) for v in range(A)]
    mean = None
    for p in parts:
        ps = jnp.sum(p, axis=1, keepdims=True)
        mean = ps if mean is None else mean + ps
    mean = mean * (1.0 / D)
    sq = None
    for p in parts:
        c = p - mean
        ps = jnp.sum(c * c, axis=1, keepdims=True)
        sq = ps if sq is None else sq + ps
    inv = 1.0 / (jnp.sqrt(sq) + 1e-5)
    out_ref[...] = jnp.concatenate([(p - mean) * inv for p in parts],
                                   axis=1)


def _ynorm_body(y_ref, s_ref, yn_ref):
    _assemble_normalize(y_ref, s_ref, BN, yn_ref)


def _normalize_y(y4, s):
    return pl.pallas_call(
        _ynorm_body,
        grid=(NJ,),
        in_specs=[
            pl.BlockSpec((1, A * MJ, C, IMW), lambda j: (j, 0, 0, 0)),
            pl.BlockSpec((IMW, IMW), lambda j: (0, 0)),
        ],
        out_specs=pl.BlockSpec((BN, D), lambda j: (j, 0)),
        out_shape=jax.ShapeDtypeStruct((N, D), jnp.float32),
    )(y4, s)


def _sim_body(x_ref, s_ref, yn_ref, idx_ref, loss_ref,
              xn_s, rmax_s, ridx_s, acc_s):
    i = pl.program_id(0)
    j = pl.program_id(1)

    @pl.when(j == 0)
    def _():
        _assemble_normalize(x_ref, s_ref, BM, xn_s)
        rmax_s[...] = jnp.full((BM, 1), NEG, jnp.float32)
        ridx_s[...] = jnp.zeros((BM, 1), jnp.int32)

    s = lax.dot_general(
        xn_s[...], yn_ref[...], (((1,), (1,)), ((), ())),
        preferred_element_type=jnp.float32,
    )
    col = j * BN + lax.broadcasted_iota(jnp.int32, (BM, BN), 1)
    bmax = jnp.max(s, axis=1, keepdims=True)
    cand = jnp.where(s == bmax, col, 2**31 - 1)
    bidx = jnp.min(cand, axis=1, keepdims=True)
    upd = bmax > rmax_s[...]
    ridx_s[...] = jnp.where(upd, bidx, ridx_s[...])
    rmax_s[...] = jnp.where(upd, bmax, rmax_s[...])

    @pl.when(j == NJ - 1)
    def _():
        idx_ref[...] = ridx_s[...]

        @pl.when(i == 0)
        def _():
            acc_s[0, 0] = 0.0

        acc_s[0, 0] += jnp.sum(1.0 - rmax_s[...])

        @pl.when(i == NI - 1)
        def _():
            loss_ref[...] = jnp.full((1, 1), acc_s[0, 0] / N, jnp.float32)


def _sim_argmax(x4, s, yn):
    return pl.pallas_call(
        _sim_body,
        grid=(NI, NJ),
        in_specs=[
            pl.BlockSpec((1, A * MI, C, IMW), lambda i, j: (i, 0, 0, 0)),
            pl.BlockSpec((IMW, IMW), lambda i, j: (0, 0)),
            pl.BlockSpec((BN, D), lambda i, j: (j, 0)),
        ],
        out_specs=[
            pl.BlockSpec((BM, 1), lambda i, j: (i, 0)),
            pl.BlockSpec((1, 1), lambda i, j: (0, 0)),
        ],
        out_shape=[
            jax.ShapeDtypeStruct((N, 1), jnp.int32),
            jax.ShapeDtypeStruct((1, 1), jnp.float32),
        ],
        scratch_shapes=[
            pltpu.VMEM((BM, D), jnp.float32),
            pltpu.VMEM((BM, 1), jnp.float32),
            pltpu.VMEM((BM, 1), jnp.int32),
            pltpu.SMEM((1, 1), jnp.float32),
        ],
        compiler_params=pltpu.CompilerParams(
            dimension_semantics=("arbitrary", "arbitrary"),
        ),
    )(x4, s, yn)


def _gather_rows(yn, idxp):
    info = plsc.get_sparse_core_info()
    nw = 28
    bpw = N // nw
    ch = 8
    nch = bpw // ch
    mesh = plsc.VectorSubcoreMesh(core_axis_name="c", subcore_axis_name="s")

    @functools.partial(
        pl.kernel, mesh=mesh,
        out_type=jax.ShapeDtypeStruct((NB, A, NB, DS), jnp.float32),
        scratch_types=[
            pltpu.VMEM((bpw,), jnp.int32),
            pltpu.VMEM((ch, D), jnp.float32),
            pltpu.VMEM((ch, D), jnp.float32),
            pltpu.SemaphoreType.DMA,
            pltpu.SemaphoreType.DMA,
            pltpu.SemaphoreType.DMA,
        ],
    )
    def k(yn_hbm, idx_hbm, out_hbm, idx_v, buf0, buf1, gsem0, gsem1, wsem):
        wid = lax.axis_index("s") * info.num_cores + lax.axis_index("c")

        @pl.when(wid < nw)
        def _():
            base = wid * bpw
            pltpu.sync_copy(idx_hbm.at[pl.ds(base, bpw)], idx_v)
            bufs = (buf0, buf1)
            gsems = (gsem0, gsem1)
            gathers = [None] * nch
            writes = [[] for _ in range(nch)]
            gathers[0] = pltpu.async_copy(
                yn_hbm.at[idx_v.at[pl.ds(0, ch)]], bufs[0], gsems[0])
            for c in range(nch):
                cur = bufs[c % 2]
                gathers[c].wait()
                if c + 1 < nch:
                    if c >= 1:
                        for h in writes[c - 1]:
                            h.wait()
                    gathers[c + 1] = pltpu.async_copy(
                        yn_hbm.at[idx_v.at[pl.ds((c + 1) * ch, ch)]],
                        bufs[(c + 1) % 2], gsems[(c + 1) % 2])
                bi = 2 * wid + (c // 7)
                bj = ch * (c % 7)
                for u in range(A):
                    writes[c].append(pltpu.make_async_copy(
                        cur.at[:, pl.ds(u * DS, DS)],
                        out_hbm.at[bi, u, pl.ds(bj, ch)],
                        wsem))
                    writes[c][-1].start()
            for c in (nch - 2, nch - 1):
                for h in writes[c]:
                    h.wait()

    return k(yn, idxp)


def kernel(x, y):
    shape = x.shape
    xt = jnp.transpose(x, (0, 1, 3, 2)).reshape(NI, A * MI, C, IMW)
    yt = jnp.transpose(y, (0, 1, 3, 2)).reshape(NJ, A * MJ, C, IMW)
    w = jnp.arange(IMW, dtype=jnp.int32)
    s = (w[:, None] == (A * (w % NB) + w // NB)[None, :]).astype(jnp.float32)
    yn = _normalize_y(yt, s)
    idx2, loss = _sim_argmax(xt, s, yn)
    new_x = _gather_rows(yn, idx2.reshape(N)).reshape(shape)
    return (loss[0, 0], new_x)


# speedup vs baseline: 2.3750x; 1.0649x over previous
import functools

import jax
import jax.numpy as jnp
from jax import lax
from jax.experimental import pallas as pl
from jax.experimental.pallas import tpu as pltpu
from jax.experimental.pallas import tpu_sc as plsc

A = 4
NB = 56
IMW = 224
C = 96
N = 3136
DS = 384
D = 1536
NPAD = 3328
BM = 784
BN = 448
MI = BM // NB
MJ = BN // NB
NI = N // BM
NJ = N // BN
NEG = -3.0e38


def _assemble_normalize(ref, s_ref, rows, out_ref):
    g = rows // NB
    val2 = ref[0].reshape(g, A, C, IMW)
    parts = []
    for u in range(A):
        vu = val2[:, u].reshape(g * C, IMW)
        qu = lax.dot_general(vu, s_ref[...], (((1,), (0,)), ((), ())),
                             precision=lax.Precision.HIGHEST,
                             preferred_element_type=jnp.float32)
        qt = jnp.transpose(qu.reshape(g, C, IMW), (0, 2, 1))
        qt = qt.reshape(g, A, NB, C)
        parts += [qt[:, v].reshape(rows, C) for v in range(A)]
    mean = None
    for p in parts:
        ps = jnp.sum(p, axis=1, keepdims=True)
        mean = ps if mean is None else mean + ps
    mean = mean * (1.0 / D)
    sq = None
    for p in parts:
        c = p - mean
        ps = jnp.sum(c * c, axis=1, keepdims=True)
        sq = ps if sq is None else sq + ps
    inv = 1.0 / (jnp.sqrt(sq) + 1e-5)
    out_ref[...] = jnp.concatenate([(p - mean) * inv for p in parts],
                                   axis=1)


def _ynorm_body(y_ref, s_ref, yn_ref):
    _assemble_normalize(y_ref, s_ref, BN, yn_ref)


def _normalize_y(y4, s):
    return pl.pallas_call(
        _ynorm_body,
        grid=(NJ,),
        in_specs=[
            pl.BlockSpec((1, A * MJ, C, IMW), lambda j: (j, 0, 0, 0)),
            pl.BlockSpec((IMW, IMW), lambda j: (0, 0)),
        ],
        out_specs=pl.BlockSpec((BN, D), lambda j: (j, 0)),
        out_shape=jax.ShapeDtypeStruct((N, D), jnp.float32),
    )(y4, s)


def _sim_body(x_ref, s_ref, yn_ref, idx_ref, loss_ref,
              xn_s, rmax_s, ridx_s, acc_s):
    i = pl.program_id(0)
    j = pl.program_id(1)

    @pl.when(j == 0)
    def _():
        _assemble_normalize(x_ref, s_ref, BM, xn_s)
        rmax_s[...] = jnp.full((BM, 1), NEG, jnp.float32)
        ridx_s[...] = jnp.zeros((BM, 1), jnp.int32)

    s = lax.dot_general(
        xn_s[...], yn_ref[...], (((1,), (1,)), ((), ())),
        preferred_element_type=jnp.float32,
    )
    col = j * BN + lax.broadcasted_iota(jnp.int32, (BM, BN), 1)
    bmax = jnp.max(s, axis=1, keepdims=True)
    cand = jnp.where(s == bmax, col, 2**31 - 1)
    bidx = jnp.min(cand, axis=1, keepdims=True)
    upd = bmax > rmax_s[...]
    ridx_s[...] = jnp.where(upd, bidx, ridx_s[...])
    rmax_s[...] = jnp.where(upd, bmax, rmax_s[...])

    @pl.when(j == NJ - 1)
    def _():
        idx_ref[...] = ridx_s[...]

        @pl.when(i == 0)
        def _():
            acc_s[0, 0] = 0.0

        acc_s[0, 0] += jnp.sum(1.0 - rmax_s[...])

        @pl.when(i == NI - 1)
        def _():
            loss_ref[...] = jnp.full((1, 1), acc_s[0, 0] / N, jnp.float32)


def _sim_argmax(x4, s, yn):
    return pl.pallas_call(
        _sim_body,
        grid=(NI, NJ),
        in_specs=[
            pl.BlockSpec((1, A * MI, C, IMW), lambda i, j: (i, 0, 0, 0)),
            pl.BlockSpec((IMW, IMW), lambda i, j: (0, 0)),
            pl.BlockSpec((BN, D), lambda i, j: (j, 0)),
        ],
        out_specs=[
            pl.BlockSpec((BM, 1), lambda i, j: (i, 0)),
            pl.BlockSpec((1, 1), lambda i, j: (0, 0)),
        ],
        out_shape=[
            jax.ShapeDtypeStruct((N, 1), jnp.int32),
            jax.ShapeDtypeStruct((1, 1), jnp.float32),
        ],
        scratch_shapes=[
            pltpu.VMEM((BM, D), jnp.float32),
            pltpu.VMEM((BM, 1), jnp.float32),
            pltpu.VMEM((BM, 1), jnp.int32),
            pltpu.SMEM((1, 1), jnp.float32),
        ],
        compiler_params=pltpu.CompilerParams(
            dimension_semantics=("arbitrary", "arbitrary"),
        ),
    )(x4, s, yn)


def _gather_rows(yn, idxp):
    info = plsc.get_sparse_core_info()
    nw = 28
    bpw = N // nw
    ch = 8
    nch = bpw // ch
    mesh = plsc.VectorSubcoreMesh(core_axis_name="c", subcore_axis_name="s")

    @functools.partial(
        pl.kernel, mesh=mesh,
        out_type=jax.ShapeDtypeStruct((NB, A, NB, DS), jnp.float32),
        scratch_types=[
            pltpu.VMEM((bpw,), jnp.int32),
            pltpu.VMEM((ch, D), jnp.float32),
            pltpu.VMEM((ch, D), jnp.float32),
            pltpu.SemaphoreType.DMA,
            pltpu.SemaphoreType.DMA,
            pltpu.SemaphoreType.DMA,
        ],
    )
    def k(yn_hbm, idx_hbm, out_hbm, idx_v, buf0, buf1, gsem0, gsem1, wsem):
        wid = lax.axis_index("s") * info.num_cores + lax.axis_index("c")

        @pl.when(wid < nw)
        def _():
            base = wid * bpw
            pltpu.sync_copy(idx_hbm.at[pl.ds(base, bpw)], idx_v)
            bufs = (buf0, buf1)
            gsems = (gsem0, gsem1)
            gathers = [None] * nch
            writes = [[] for _ in range(nch)]
            gathers[0] = pltpu.async_copy(
                yn_hbm.at[idx_v.at[pl.ds(0, ch)]], bufs[0], gsems[0])
            for c in range(nch):
                cur = bufs[c % 2]
                gathers[c].wait()
                if c + 1 < nch:
                    if c >= 1:
                        for h in writes[c - 1]:
                            h.wait()
                    gathers[c + 1] = pltpu.async_copy(
                        yn_hbm.at[idx_v.at[pl.ds((c + 1) * ch, ch)]],
                        bufs[(c + 1) % 2], gsems[(c + 1) % 2])
                bi = 2 * wid + (c // 7)
                bj = ch * (c % 7)
                for u in range(A):
                    writes[c].append(pltpu.make_async_copy(
                        cur.at[:, pl.ds(u * DS, DS)],
                        out_hbm.at[bi, u, pl.ds(bj, ch)],
                        wsem))
                    writes[c][-1].start()
            for c in (nch - 2, nch - 1):
                for h in writes[c]:
                    h.wait()

    return k(yn, idxp)


def kernel(x, y):
    shape = x.shape
    xt = jnp.transpose(x, (0, 1, 3, 2)).reshape(NI, A * MI, C, IMW)
    yt = jnp.transpose(y, (0, 1, 3, 2)).reshape(NJ, A * MJ, C, IMW)
    w = jnp.arange(IMW, dtype=jnp.int32)
    s = (w[:, None] == (A * (w % NB) + w // NB)[None, :]).astype(jnp.float32)
    yn = _normalize_y(yt, s)
    idx2, loss = _sim_argmax(xt, s, yn)
    new_x = _gather_rows(yn, idx2.reshape(N)).reshape(shape)
    return (loss[0, 0], new_x)
